# Initial kernel scaffold; baseline (speedup 1.0000x reference)
import jax
import jax.numpy as jnp
from jax.experimental import pallas as pl


def kernel(x, edge_index, batch, g1_W0, g1_b0, g1_W1, g1_b1, g1_W2, g1_b2, g1_ln_g, g1_ln_b, g1_W3, g1_b3, g2_W0, g2_b0, g2_W1, g2_b1, g2_W2, g2_b2, g2_ln_g, g2_ln_b, g2_W3, g2_b3, g3_W0, g3_b0, g3_W1, g3_b1, g3_W2, g3_b2, g3_ln_g, g3_ln_b, g3_W3, g3_b3, f_W0, f_b0, f_W1, f_b1, f_W2, f_b2, f_ln_g, f_ln_b, f_W3, f_b3):
    raise NotImplementedError("write your pallas kernel here")


# speedup vs baseline: 11.5963x; 11.5963x over previous
import jax
import jax.numpy as jnp
from jax import lax
from jax.experimental import pallas as pl
from jax.experimental.pallas import tpu as pltpu
from jax.experimental.pallas import tpu_sc as plsc

_N = 10000
_E = 320000
_D = 128
_H = 64
_G = 128
_NP = 10240
_BLK = 1024
_NBLK = _NP // _BLK
_NC = 2
_NS = 16
_NT = _NC * _NS
_EPT = _E // _NT
_L = 16


def _hist_body(dst_hbm, out_hbm, idx_v, counts_v):
    wid = lax.axis_index("s") * _NC + lax.axis_index("c")
    base = wid * _EPT
    pltpu.sync_copy(dst_hbm.at[pl.ds(base, _EPT)], idx_v)

    zero = jnp.zeros((_L,), jnp.float32)

    def zbody(i, c):
        counts_v[pl.ds(i * _L, _L)] = zero
        return c

    lax.fori_loop(0, _NP // _L, zbody, 0, unroll=8)

    one = jnp.ones((_L,), jnp.float32)

    def sbody(i, c):
        idx = idx_v[pl.ds(i * _L, _L)]
        plsc.addupdate_scatter(counts_v, [idx], one)
        return c

    lax.fori_loop(0, _EPT // _L, sbody, 0, unroll=8)
    pltpu.sync_copy(counts_v, out_hbm.at[wid])


def _sc_counts(dst):
    mesh = plsc.VectorSubcoreMesh(
        core_axis_name="c", subcore_axis_name="s",
        num_cores=_NC, num_subcores=_NS)
    return pl.kernel(
        _hist_body,
        out_type=jax.ShapeDtypeStruct((_NT, _NP), jnp.float32),
        mesh=mesh,
        scratch_types=[
            pltpu.VMEM((_EPT,), jnp.int32),
            pltpu.VMEM((_NP,), jnp.float32),
        ],
        compiler_params=pltpu.CompilerParams(needs_layout_passes=False),
    )(dst)


def _mlp(h, W0, b0, W1, b1, W2, b2, g, b, W3, b3):
    h = jnp.maximum(jnp.dot(h, W0, preferred_element_type=jnp.float32) + b0, 0.0)
    h = jnp.maximum(jnp.dot(h, W1, preferred_element_type=jnp.float32) + b1, 0.0)
    h = jnp.maximum(jnp.dot(h, W2, preferred_element_type=jnp.float32) + b2, 0.0)
    mu = jnp.mean(h, axis=-1, keepdims=True)
    var = jnp.mean((h - mu) * (h - mu), axis=-1, keepdims=True)
    h = (h - mu) * lax.rsqrt(var + 1e-5) * g + b
    return jnp.dot(h, W3, preferred_element_type=jnp.float32) + b3


def _stage_a(x_ref, cp_ref, *rest):
    w = [r[...] for r in rest[0:10]]
    x1_ref, ns_ref = rest[10], rest[11]
    i = pl.program_id(0)
    x1 = _mlp(x_ref[...], *w)
    row = lax.broadcasted_iota(jnp.int32, (_BLK, 1), 0) + i * _BLK
    x1 = jnp.where(row < _N, x1, 0.0)
    x1_ref[...] = x1
    counts = jnp.sum(cp_ref[...], axis=0, keepdims=True)
    part = jnp.dot(counts, x1, preferred_element_type=jnp.float32)

    @pl.when(i == 0)
    def _():
        ns_ref[...] = part

    @pl.when(i > 0)
    def _():
        ns_ref[...] = ns_ref[...] + part


def _stage_b(x1_ref, ns_ref, b_ref, *rest):
    g2 = [r[...] for r in rest[0:10]]
    g3 = [r[...] for r in rest[10:20]]
    fw = [r[...] for r in rest[20:30]]
    out_ref, pooled = rest[30], rest[31]

    i = pl.program_id(0)
    xe = _mlp(ns_ref[...], *g2)
    x2 = _mlp(xe + x1_ref[...], *g3)
    bids = b_ref[0]
    gid = lax.broadcasted_iota(jnp.int32, (_G, _BLK), 0)
    oh = jnp.where(gid == bids, 1.0, 0.0)
    contrib = jnp.dot(oh, x2, preferred_element_type=jnp.float32)

    @pl.when(i == 0)
    def _():
        pooled[...] = contrib

    @pl.when(i > 0)
    def _():
        pooled[...] = pooled[...] + contrib

    @pl.when(i == _NBLK - 1)
    def _():
        out_ref[...] = _mlp(pooled[...], *fw)


def _full(shape):
    nd = len(shape)
    return pl.BlockSpec(shape, lambda i, _nd=nd: (0,) * _nd)


def kernel(x, edge_index, batch,
           g1_W0, g1_b0, g1_W1, g1_b1, g1_W2, g1_b2, g1_ln_g, g1_ln_b, g1_W3, g1_b3,
           g2_W0, g2_b0, g2_W1, g2_b1, g2_W2, g2_b2, g2_ln_g, g2_ln_b, g2_W3, g2_b3,
           g3_W0, g3_b0, g3_W1, g3_b1, g3_W2, g3_b2, g3_ln_g, g3_ln_b, g3_W3, g3_b3,
           f_W0, f_b0, f_W1, f_b1, f_W2, f_b2, f_ln_g, f_ln_b, f_W3, f_b3):
    r = lambda v: v.reshape(1, -1)
    g1 = (g1_W0, r(g1_b0), g1_W1, r(g1_b1), g1_W2, r(g1_b2),
          r(g1_ln_g), r(g1_ln_b), g1_W3, r(g1_b3))
    g2 = (g2_W0, r(g2_b0), g2_W1, r(g2_b1), g2_W2, r(g2_b2),
          r(g2_ln_g), r(g2_ln_b), g2_W3, r(g2_b3))
    g3 = (g3_W0, r(g3_b0), g3_W1, r(g3_b1), g3_W2, r(g3_b2),
          r(g3_ln_g), r(g3_ln_b), g3_W3, r(g3_b3))
    fw = (f_W0, r(f_b0), f_W1, r(f_b1), f_W2, r(f_b2),
          r(f_ln_g), r(f_ln_b), f_W3, r(f_b3))

    cp = _sc_counts(edge_index[1])
    batch_p = jnp.pad(batch, (0, _NP - _N),
                      constant_values=_G).reshape(_NBLK, 1, _BLK)

    x1, ns = pl.pallas_call(
        _stage_a,
        grid=(_NBLK,),
        in_specs=[
            pl.BlockSpec((_BLK, _D), lambda i: (i, 0)),
            pl.BlockSpec((_NT, _BLK), lambda i: (0, i)),
        ] + [_full(w.shape) for w in g1],
        out_specs=[
            pl.BlockSpec((_BLK, _H), lambda i: (i, 0)),
            pl.BlockSpec((1, _H), lambda i: (0, 0)),
        ],
        out_shape=[
            jax.ShapeDtypeStruct((_NP, _H), jnp.float32),
            jax.ShapeDtypeStruct((1, _H), jnp.float32),
        ],
    )(x, cp, *g1)

    out = pl.pallas_call(
        _stage_b,
        grid=(_NBLK,),
        in_specs=[
            pl.BlockSpec((_BLK, _H), lambda i: (i, 0)),
            pl.BlockSpec((1, _H), lambda i: (0, 0)),
            pl.BlockSpec((1, 1, _BLK), lambda i: (i, 0, 0)),
        ] + [_full(w.shape) for w in (g2 + g3 + fw)],
        out_specs=pl.BlockSpec((_G, 1), lambda i: (0, 0)),
        out_shape=jax.ShapeDtypeStruct((_G, 1), jnp.float32),
        scratch_shapes=[pltpu.VMEM((_G, _H), jnp.float32)],
    )(x1, ns, batch_p, *g2, *g3, *fw)

    return out


# speedup vs baseline: 15.4549x; 1.3328x over previous
import jax
import jax.numpy as jnp
from jax import lax
from jax.experimental import pallas as pl
from jax.experimental.pallas import tpu as pltpu
from jax.experimental.pallas import tpu_sc as plsc

_N = 10000
_E = 320000
_D = 128
_H = 64
_G = 128
_NP = 10240
_BLK = 1024
_NBLK = _NP // _BLK
_NC = 2
_NS = 16
_NT = _NC * _NS
_EPT = _E // _NT
_L = 16


def _hist_body(edge_hbm, out_hbm, idx_v, counts_v):
    wid = lax.axis_index("s") * _NC + lax.axis_index("c")
    base = wid * _EPT
    pltpu.sync_copy(edge_hbm.at[pl.ds(_E + base, _EPT)], idx_v)

    zero = jnp.zeros((_L,), jnp.float32)

    def zbody(i, c):
        counts_v[pl.ds(i * _L, _L)] = zero
        return c

    lax.fori_loop(0, _NP // _L, zbody, 0, unroll=8)

    one = jnp.ones((_L,), jnp.float32)

    def sbody(i, c):
        idx = idx_v[pl.ds(i * _L, _L)]
        plsc.addupdate_scatter(counts_v, [idx], one)
        return c

    lax.fori_loop(0, _EPT // _L, sbody, 0, unroll=8)
    pltpu.sync_copy(counts_v, out_hbm.at[wid])


def _sc_counts(edge_index):
    mesh = plsc.VectorSubcoreMesh(
        core_axis_name="c", subcore_axis_name="s",
        num_cores=_NC, num_subcores=_NS)
    return pl.kernel(
        _hist_body,
        out_type=jax.ShapeDtypeStruct((_NT, _NP), jnp.float32),
        mesh=mesh,
        scratch_types=[
            pltpu.VMEM((_EPT,), jnp.int32),
            pltpu.VMEM((_NP,), jnp.float32),
        ],
        compiler_params=pltpu.CompilerParams(needs_layout_passes=False),
    )(edge_index.reshape(-1))


def _mlp(h, W0, b0, W1, b1, W2, b2, g, b, W3, b3):
    h = jnp.maximum(jnp.dot(h, W0, preferred_element_type=jnp.float32) + b0, 0.0)
    h = jnp.maximum(jnp.dot(h, W1, preferred_element_type=jnp.float32) + b1, 0.0)
    h = jnp.maximum(jnp.dot(h, W2, preferred_element_type=jnp.float32) + b2, 0.0)
    mu = jnp.mean(h, axis=-1, keepdims=True)
    var = jnp.mean((h - mu) * (h - mu), axis=-1, keepdims=True)
    h = (h - mu) * lax.rsqrt(var + 1e-5) * g + b
    return jnp.dot(h, W3, preferred_element_type=jnp.float32) + b3


def _fused(x_ref, cp_ref, b_ref, *rest):
    g1, g2, g3, fw = rest[0:10], rest[10:20], rest[20:30], rest[30:40]
    out_ref = rest[40]
    x1_buf, ns_ref, xe_ref, pooled = rest[41], rest[42], rest[43], rest[44]
    p = pl.program_id(0)
    i = pl.program_id(1)
    off = pl.multiple_of(i * _BLK, _BLK)

    @pl.when(p == 0)
    def _phase0():
        x1 = _mlp(x_ref[...], *[r[...] for r in g1])
        row = lax.broadcasted_iota(jnp.int32, (_BLK, 1), 0) + i * _BLK
        x1 = jnp.where(row < _N, x1, 0.0)
        x1_buf[pl.ds(off, _BLK), :] = x1
        counts = jnp.sum(cp_ref[...], axis=0, keepdims=True)
        part = jnp.dot(counts, x1, preferred_element_type=jnp.float32)

        @pl.when(i == 0)
        def _():
            ns_ref[...] = part

        @pl.when(i > 0)
        def _():
            ns_ref[...] = ns_ref[...] + part

    @pl.when(p == 1)
    def _phase1():
        @pl.when(i == 0)
        def _():
            xe_ref[...] = _mlp(ns_ref[...], *[r[...] for r in g2])

        x1 = x1_buf[pl.ds(off, _BLK), :]
        x2 = _mlp(xe_ref[...] + x1, *[r[...] for r in g3])
        bids = b_ref[0]
        gid = lax.broadcasted_iota(jnp.int32, (_G, _BLK), 0)
        oh = jnp.where(gid == bids, 1.0, 0.0)
        contrib = jnp.dot(oh, x2, preferred_element_type=jnp.float32)

        @pl.when(i == 0)
        def _():
            pooled[...] = contrib

        @pl.when(i > 0)
        def _():
            pooled[...] = pooled[...] + contrib

        @pl.when(i == _NBLK - 1)
        def _():
            out_ref[...] = _mlp(pooled[...], *[r[...] for r in fw])


def _full(shape):
    nd = len(shape)
    return pl.BlockSpec(shape, lambda p, i, _nd=nd: (0,) * _nd)


def kernel(x, edge_index, batch,
           g1_W0, g1_b0, g1_W1, g1_b1, g1_W2, g1_b2, g1_ln_g, g1_ln_b, g1_W3, g1_b3,
           g2_W0, g2_b0, g2_W1, g2_b1, g2_W2, g2_b2, g2_ln_g, g2_ln_b, g2_W3, g2_b3,
           g3_W0, g3_b0, g3_W1, g3_b1, g3_W2, g3_b2, g3_ln_g, g3_ln_b, g3_W3, g3_b3,
           f_W0, f_b0, f_W1, f_b1, f_W2, f_b2, f_ln_g, f_ln_b, f_W3, f_b3):
    r = lambda v: v.reshape(1, -1)
    g1 = (g1_W0, r(g1_b0), g1_W1, r(g1_b1), g1_W2, r(g1_b2),
          r(g1_ln_g), r(g1_ln_b), g1_W3, r(g1_b3))
    g2 = (g2_W0, r(g2_b0), g2_W1, r(g2_b1), g2_W2, r(g2_b2),
          r(g2_ln_g), r(g2_ln_b), g2_W3, r(g2_b3))
    g3 = (g3_W0, r(g3_b0), g3_W1, r(g3_b1), g3_W2, r(g3_b2),
          r(g3_ln_g), r(g3_ln_b), g3_W3, r(g3_b3))
    fw = (f_W0, r(f_b0), f_W1, r(f_b1), f_W2, r(f_b2),
          r(f_ln_g), r(f_ln_b), f_W3, r(f_b3))

    cp = _sc_counts(edge_index)
    batch_p = jnp.pad(batch, (0, _NP - _N),
                      constant_values=_G).reshape(_NBLK, 1, _BLK)

    out = pl.pallas_call(
        _fused,
        grid=(2, _NBLK),
        in_specs=[
            pl.BlockSpec((_BLK, _D), lambda p, i: ((1 - p) * i, 0)),
            pl.BlockSpec((_NT, _BLK), lambda p, i: (0, (1 - p) * i)),
            pl.BlockSpec((1, 1, _BLK), lambda p, i: (p * i, 0, 0)),
        ] + [_full(w.shape) for w in (g1 + g2 + g3 + fw)],
        out_specs=pl.BlockSpec((_G, 1), lambda p, i: (0, 0)),
        out_shape=jax.ShapeDtypeStruct((_G, 1), jnp.float32),
        scratch_shapes=[
            pltpu.VMEM((_NP, _H), jnp.float32),
            pltpu.VMEM((1, _H), jnp.float32),
            pltpu.VMEM((1, _H), jnp.float32),
            pltpu.VMEM((_G, _H), jnp.float32),
        ],
    )(x, cp, batch_p, *g1, *g2, *g3, *fw)

    return out


# speedup vs baseline: 17.7953x; 1.1514x over previous
import jax
import jax.numpy as jnp
from jax import lax
from jax.experimental import pallas as pl
from jax.experimental.pallas import tpu as pltpu
from jax.experimental.pallas import tpu_sc as plsc

_N = 10000
_E = 320000
_D = 128
_H = 64
_G = 128
_NP = 10240
_BLK = 2048
_NBLK = _NP // _BLK
_NC = 2
_NS = 16
_NT = _NC * _NS
_EPT = _E // _NT
_L = 16


def _hist_body(edge_hbm, out_hbm, idx_v, counts_v):
    wid = lax.axis_index("s") * _NC + lax.axis_index("c")
    base = wid * _EPT
    pltpu.sync_copy(edge_hbm.at[pl.ds(_E + base, _EPT)], idx_v)

    zero = jnp.zeros((_L,), jnp.float32)

    def zbody(i, c):
        counts_v[pl.ds(i * _L, _L)] = zero
        return c

    lax.fori_loop(0, _NP // _L, zbody, 0, unroll=8)

    one = jnp.ones((_L,), jnp.float32)

    def sbody(i, c):
        idx = idx_v[pl.ds(i * _L, _L)]
        plsc.addupdate_scatter(counts_v, [idx], one)
        return c

    lax.fori_loop(0, _EPT // _L, sbody, 0, unroll=8)
    pltpu.sync_copy(counts_v, out_hbm.at[wid])


def _sc_counts(edge_index):
    mesh = plsc.VectorSubcoreMesh(
        core_axis_name="c", subcore_axis_name="s",
        num_cores=_NC, num_subcores=_NS)
    return pl.kernel(
        _hist_body,
        out_type=jax.ShapeDtypeStruct((_NT, _NP), jnp.float32),
        mesh=mesh,
        scratch_types=[
            pltpu.VMEM((_EPT,), jnp.int32),
            pltpu.VMEM((_NP,), jnp.float32),
        ],
        compiler_params=pltpu.CompilerParams(needs_layout_passes=False),
    )(edge_index.reshape(-1))


def _mlp(h, W0, b0, W1, b1, W2, b2, g, b, W3, b3):
    h = jnp.maximum(jnp.dot(h, W0, preferred_element_type=jnp.float32) + b0, 0.0)
    h = jnp.maximum(jnp.dot(h, W1, preferred_element_type=jnp.float32) + b1, 0.0)
    h = jnp.maximum(jnp.dot(h, W2, preferred_element_type=jnp.float32) + b2, 0.0)
    mu = jnp.mean(h, axis=-1, keepdims=True)
    var = jnp.mean((h - mu) * (h - mu), axis=-1, keepdims=True)
    h = (h - mu) * lax.rsqrt(var + 1e-5) * g + b
    return jnp.dot(h, W3, preferred_element_type=jnp.float32) + b3


def _fused(x_ref, cp_ref, b_ref, *rest):
    g1, g2, g3, fw = rest[0:10], rest[10:20], rest[20:30], rest[30:40]
    out_ref = rest[40]
    x1_buf, ns_ref, xe_ref, pooled = rest[41], rest[42], rest[43], rest[44]
    p = pl.program_id(0)
    i = pl.program_id(1)
    off = pl.multiple_of(i * _BLK, _BLK)

    @pl.when(p == 0)
    def _phase0():
        x1 = _mlp(x_ref[...], *[r[...] for r in g1])
        row = lax.broadcasted_iota(jnp.int32, (_BLK, 1), 0) + i * _BLK
        x1 = jnp.where(row < _N, x1, 0.0)
        x1_buf[pl.ds(off, _BLK), :] = x1
        counts = jnp.sum(cp_ref[...], axis=0, keepdims=True)
        part = jnp.dot(counts, x1, preferred_element_type=jnp.float32)

        @pl.when(i == 0)
        def _():
            ns_ref[...] = part

        @pl.when(i > 0)
        def _():
            ns_ref[...] = ns_ref[...] + part

    @pl.when(p == 1)
    def _phase1():
        @pl.when(i == 0)
        def _():
            xe_ref[...] = _mlp(ns_ref[...], *[r[...] for r in g2])

        x1 = x1_buf[pl.ds(off, _BLK), :]
        x2 = _mlp(xe_ref[...] + x1, *[r[...] for r in g3])
        bids = b_ref[0]
        gid = lax.broadcasted_iota(jnp.int32, (_G, _BLK), 0)
        oh = jnp.where(gid == bids, 1.0, 0.0)
        contrib = jnp.dot(oh, x2, preferred_element_type=jnp.float32)

        @pl.when(i == 0)
        def _():
            pooled[...] = contrib

        @pl.when(i > 0)
        def _():
            pooled[...] = pooled[...] + contrib

        @pl.when(i == _NBLK - 1)
        def _():
            out_ref[...] = _mlp(pooled[...], *[r[...] for r in fw])


def _full(shape):
    nd = len(shape)
    return pl.BlockSpec(shape, lambda p, i, _nd=nd: (0,) * _nd)


def kernel(x, edge_index, batch,
           g1_W0, g1_b0, g1_W1, g1_b1, g1_W2, g1_b2, g1_ln_g, g1_ln_b, g1_W3, g1_b3,
           g2_W0, g2_b0, g2_W1, g2_b1, g2_W2, g2_b2, g2_ln_g, g2_ln_b, g2_W3, g2_b3,
           g3_W0, g3_b0, g3_W1, g3_b1, g3_W2, g3_b2, g3_ln_g, g3_ln_b, g3_W3, g3_b3,
           f_W0, f_b0, f_W1, f_b1, f_W2, f_b2, f_ln_g, f_ln_b, f_W3, f_b3):
    r = lambda v: v.reshape(1, -1)
    g1 = (g1_W0, r(g1_b0), g1_W1, r(g1_b1), g1_W2, r(g1_b2),
          r(g1_ln_g), r(g1_ln_b), g1_W3, r(g1_b3))
    g2 = (g2_W0, r(g2_b0), g2_W1, r(g2_b1), g2_W2, r(g2_b2),
          r(g2_ln_g), r(g2_ln_b), g2_W3, r(g2_b3))
    g3 = (g3_W0, r(g3_b0), g3_W1, r(g3_b1), g3_W2, r(g3_b2),
          r(g3_ln_g), r(g3_ln_b), g3_W3, r(g3_b3))
    fw = (f_W0, r(f_b0), f_W1, r(f_b1), f_W2, r(f_b2),
          r(f_ln_g), r(f_ln_b), f_W3, r(f_b3))

    cp = _sc_counts(edge_index)
    batch_p = jnp.pad(batch, (0, _NP - _N),
                      constant_values=_G).reshape(_NBLK, 1, _BLK)

    out = pl.pallas_call(
        _fused,
        grid=(2, _NBLK),
        in_specs=[
            pl.BlockSpec((_BLK, _D), lambda p, i: ((1 - p) * i, 0)),
            pl.BlockSpec((_NT, _BLK), lambda p, i: (0, (1 - p) * i)),
            pl.BlockSpec((1, 1, _BLK), lambda p, i: (p * i, 0, 0)),
        ] + [_full(w.shape) for w in (g1 + g2 + g3 + fw)],
        out_specs=pl.BlockSpec((_G, 1), lambda p, i: (0, 0)),
        out_shape=jax.ShapeDtypeStruct((_G, 1), jnp.float32),
        scratch_shapes=[
            pltpu.VMEM((_NP, _H), jnp.float32),
            pltpu.VMEM((1, _H), jnp.float32),
            pltpu.VMEM((1, _H), jnp.float32),
            pltpu.VMEM((_G, _H), jnp.float32),
        ],
    )(x, cp, batch_p, *g1, *g2, *g3, *fw)

    return out
